# baseline (speedup 1.0000x reference)
import functools

import jax
import jax.numpy as jnp
from jax import lax
from jax.experimental import pallas as pl
from jax.experimental.pallas import tpu as pltpu
from jax.experimental.pallas import tpu_sc as plsc

B = 4096
L = 200
H = 64
TYPE_VOCAB = 4
EPS = 1e-12

BL = B * L
NW = 32
ROWS_PER_W = BL // NW
CHUNK = 1024
NSUB = CHUNK // 128
NCHUNK = ROWS_PER_W // CHUNK


def _rsqrt(x):
    i = plsc.bitcast(x, jnp.int32)
    i = jnp.int32(0x5F3759DF) - (i >> 1)
    y = plsc.bitcast(i, jnp.float32)
    for _ in range(3):
        y = y * (1.5 - 0.5 * x * y * y)
    return y


def _body(seq_hbm, tid_hbm, table_hbm, type_hbm, gamma_hbm, beta_hbm,
          out_hbm, idx_v, tid_v, rows_v, type_v, gamma_v, beta_v, sem):
    wid = lax.axis_index("s") * 2 + lax.axis_index("c")
    base = wid * ROWS_PER_W

    pltpu.sync_copy(type_hbm, type_v)
    pltpu.sync_copy(gamma_hbm, gamma_v)
    pltpu.sync_copy(beta_hbm, beta_v)

    lane = lax.iota(jnp.int32, 16)
    inv_h = jnp.float32(1.0 / H)

    def chunk_body(g, carry):
        start = base + g * CHUNK
        row0 = pl.multiple_of(start // 128, 8)
        pltpu.sync_copy(seq_hbm.at[pl.ds(row0, NSUB)], idx_v)
        pltpu.sync_copy(tid_hbm.at[pl.ds(start, CHUNK)], tid_v)
        cps = [
            pltpu.async_copy(table_hbm.at[idx_v.at[j]],
                             rows_v.at[pl.ds(j * 128, 128)], sem)
            for j in range(NSUB)
        ]
        for cp in cps:
            cp.wait()

        def group_body(gi, c2):
            r0 = gi * 16
            row_ids = r0 + lane
            tbase = tid_v[pl.ds(r0, 16)] * H
            s = jnp.zeros((16,), jnp.float32)
            ss = jnp.zeros((16,), jnp.float32)
            for h in range(H):
                hh = jnp.full((16,), h, jnp.int32)
                v = plsc.load_gather(rows_v, [row_ids, hh])
                t = plsc.load_gather(type_v, [tbase + h])
                val = v + t
                plsc.store_scatter(rows_v, [row_ids, hh], val)
                s = s + val
                ss = ss + val * val
            mean = s * inv_h
            var = ss * inv_h - mean * mean
            rstd = _rsqrt(var + EPS)
            for h in range(H):
                hh = jnp.full((16,), h, jnp.int32)
                val = plsc.load_gather(rows_v, [row_ids, hh])
                gm = plsc.load_gather(gamma_v, [hh])
                bt = plsc.load_gather(beta_v, [hh])
                o = (val - mean) * rstd * gm + bt
                plsc.store_scatter(rows_v, [row_ids, hh], o)
            return c2

        lax.fori_loop(0, CHUNK // 16, group_body, 0)
        pltpu.sync_copy(rows_v, out_hbm.at[pl.ds(start, CHUNK)])
        return carry

    lax.fori_loop(0, NCHUNK, chunk_body, 0)


@jax.jit
def _run(seq2d, tid_flat, seq_table, type_flat, gamma, beta):
    mesh = plsc.VectorSubcoreMesh(core_axis_name="c", subcore_axis_name="s")
    k = pl.kernel(
        _body,
        out_type=jax.ShapeDtypeStruct((BL, H), jnp.float32),
        mesh=mesh,
        scratch_types=[
            pltpu.VMEM((NSUB, 128), jnp.int32),
            pltpu.VMEM((CHUNK,), jnp.int32),
            pltpu.VMEM((CHUNK, H), jnp.float32),
            pltpu.VMEM((TYPE_VOCAB * H,), jnp.float32),
            pltpu.VMEM((H,), jnp.float32),
            pltpu.VMEM((H,), jnp.float32),
            pltpu.SemaphoreType.DMA,
        ],
        compiler_params=pltpu.CompilerParams(
            use_tc_tiling_on_sc=False,
            needs_layout_passes=False,
        ),
    )
    return k(seq2d, tid_flat, seq_table, type_flat, gamma, beta)


def kernel(seq, type_ids, seq_table, type_table, gamma, beta):
    seq2d = seq.astype(jnp.int32).reshape(BL // 128, 128)
    tid_flat = type_ids.astype(jnp.int32).reshape(BL)
    type_flat = type_table.reshape(TYPE_VOCAB * H)
    out = _run(seq2d, tid_flat, seq_table, type_flat, gamma, beta)
    return out.reshape(B, L, H)


# speedup vs baseline: 2.1392x; 2.1392x over previous
import functools

import jax
import jax.numpy as jnp
from jax import lax
from jax.experimental import pallas as pl
from jax.experimental.pallas import tpu as pltpu
from jax.experimental.pallas import tpu_sc as plsc

B = 4096
L = 200
H = 64
TYPE_VOCAB = 4
EPS = 1e-12

BL = B * L
NW = 32
ROWS_PER_W = BL // NW
CHUNK = 1024
NSUB = CHUNK // 128
NCHUNK = ROWS_PER_W // CHUNK


def _rsqrt(x):
    i = plsc.bitcast(x, jnp.int32)
    i = jnp.int32(0x5F3759DF) - (i >> 1)
    y = plsc.bitcast(i, jnp.float32)
    for _ in range(3):
        y = y * (1.5 - 0.5 * x * y * y)
    return y


def _body(seq_hbm, tid_hbm, table_hbm, type_hbm, gamma_hbm, beta_hbm,
          out_hbm, idx_v, tid_v, rows_v, type_v, gamma_v, beta_v, sem):
    wid = lax.axis_index("s") * 2 + lax.axis_index("c")
    base = wid * ROWS_PER_W

    pltpu.sync_copy(type_hbm, type_v)
    pltpu.sync_copy(gamma_hbm, gamma_v)
    pltpu.sync_copy(beta_hbm, beta_v)

    lane = lax.iota(jnp.int32, 16)
    inv_h = jnp.float32(1.0 / H)

    def chunk_body(g, carry):
        start = base + g * CHUNK
        row0 = pl.multiple_of(start // 128, 8)
        pltpu.sync_copy(seq_hbm.at[pl.ds(row0, NSUB)], idx_v)
        pltpu.sync_copy(tid_hbm.at[pl.ds(start, CHUNK)], tid_v)
        cps = [
            pltpu.async_copy(table_hbm.at[idx_v.at[j]],
                             rows_v.at[pl.ds(j * 128, 128)], sem)
            for j in range(NSUB)
        ]
        for cp in cps:
            cp.wait()

        def group_body(gi, c2):
            r0 = gi * 16
            row_ids = r0 + lane
            tbase = tid_v[pl.ds(r0, 16)] * H
            s = jnp.zeros((16,), jnp.float32)
            ss = jnp.zeros((16,), jnp.float32)
            hh = lane
            for _ in range(H):
                v = plsc.load_gather(rows_v, [row_ids, hh])
                t = plsc.load_gather(type_v, [tbase + hh])
                val = v + t
                plsc.store_scatter(rows_v, [row_ids, hh], val)
                s = s + val
                ss = ss + val * val
                hh = (hh + 1) & (H - 1)
            mean = s * inv_h
            var = ss * inv_h - mean * mean
            rstd = _rsqrt(var + EPS)
            hh = lane
            for _ in range(H):
                val = plsc.load_gather(rows_v, [row_ids, hh])
                gm = plsc.load_gather(gamma_v, [hh])
                bt = plsc.load_gather(beta_v, [hh])
                o = (val - mean) * rstd * gm + bt
                plsc.store_scatter(rows_v, [row_ids, hh], o)
                hh = (hh + 1) & (H - 1)
            return c2

        lax.fori_loop(0, CHUNK // 16, group_body, 0)
        pltpu.sync_copy(rows_v, out_hbm.at[pl.ds(start, CHUNK)])
        return carry

    lax.fori_loop(0, NCHUNK, chunk_body, 0)


@jax.jit
def _run(seq2d, tid_flat, seq_table, type_flat, gamma, beta):
    mesh = plsc.VectorSubcoreMesh(core_axis_name="c", subcore_axis_name="s")
    k = pl.kernel(
        _body,
        out_type=jax.ShapeDtypeStruct((BL, H), jnp.float32),
        mesh=mesh,
        scratch_types=[
            pltpu.VMEM((NSUB, 128), jnp.int32),
            pltpu.VMEM((CHUNK,), jnp.int32),
            pltpu.VMEM((CHUNK, H), jnp.float32),
            pltpu.VMEM((TYPE_VOCAB * H,), jnp.float32),
            pltpu.VMEM((H,), jnp.float32),
            pltpu.VMEM((H,), jnp.float32),
            pltpu.SemaphoreType.DMA,
        ],
        compiler_params=pltpu.CompilerParams(
            use_tc_tiling_on_sc=False,
            needs_layout_passes=False,
        ),
    )
    return k(seq2d, tid_flat, seq_table, type_flat, gamma, beta)


def kernel(seq, type_ids, seq_table, type_table, gamma, beta):
    seq2d = seq.astype(jnp.int32).reshape(BL // 128, 128)
    tid_flat = type_ids.astype(jnp.int32).reshape(BL)
    type_flat = type_table.reshape(TYPE_VOCAB * H)
    out = _run(seq2d, tid_flat, seq_table, type_flat, gamma, beta)
    return out.reshape(B, L, H)


# speedup vs baseline: 3.3807x; 1.5803x over previous
import functools

import jax
import jax.numpy as jnp
from jax import lax
from jax.experimental import pallas as pl
from jax.experimental.pallas import tpu as pltpu
from jax.experimental.pallas import tpu_sc as plsc

B = 4096
L = 200
H = 64
TYPE_VOCAB = 4
EPS = 1e-12

BL = B * L
NW = 32
ROWS_PER_W = BL // NW
CHUNK = 1024
NSUB = CHUNK // 128
NCHUNK = ROWS_PER_W // CHUNK


def _rsqrt(x):
    i = plsc.bitcast(x, jnp.int32)
    i = jnp.int32(0x5F3759DF) - (i >> 1)
    y = plsc.bitcast(i, jnp.float32)
    for _ in range(3):
        y = y * (1.5 - 0.5 * x * y * y)
    return y


def _body(seq_hbm, tid_hbm, table_hbm, type_hbm, gamma_hbm, beta_hbm,
          out_hbm, idx_v, tid_v, rows_v, type_v, gamma_v, beta_v, sem):
    wid = lax.axis_index("s") * 2 + lax.axis_index("c")
    base = wid * ROWS_PER_W

    pltpu.sync_copy(type_hbm, type_v)
    pltpu.sync_copy(gamma_hbm, gamma_v)
    pltpu.sync_copy(beta_hbm, beta_v)

    lane = lax.iota(jnp.int32, 16)
    inv_h = jnp.float32(1.0 / H)

    def chunk_body(g, carry):
        start = base + g * CHUNK
        row0 = pl.multiple_of(start // 128, 8)
        pltpu.sync_copy(seq_hbm.at[pl.ds(row0, NSUB)], idx_v)
        pltpu.sync_copy(tid_hbm.at[pl.ds(start, CHUNK)], tid_v)
        cps = [
            pltpu.async_copy(table_hbm.at[idx_v.at[j]],
                             rows_v.at[pl.ds(j * 128, 128)], sem)
            for j in range(NSUB)
        ]
        for cp in cps:
            cp.wait()

        def group_body(gi, c2):
            r0 = gi * 16
            row_ids = r0 + lane
            tbase = tid_v[pl.ds(r0, 16)] * H
            acc = [jnp.zeros((16,), jnp.float32) for _ in range(4)]
            accq = [jnp.zeros((16,), jnp.float32) for _ in range(4)]
            hh = lane
            for d in range(H):
                v = plsc.load_gather(rows_v, [row_ids, hh])
                t = plsc.load_gather(type_v, [tbase + hh])
                val = v + t
                k = d & 3
                acc[k] = acc[k] + val
                accq[k] = accq[k] + val * val
                hh = (hh + 1) & (H - 1)
            mean = ((acc[0] + acc[1]) + (acc[2] + acc[3])) * inv_h
            ssum = (accq[0] + accq[1]) + (accq[2] + accq[3])
            var = ssum * inv_h - mean * mean
            rstd = _rsqrt(var + EPS)
            hh = lane
            for _ in range(H // 8):
                vs, ts, gs, bs, hs = [], [], [], [], []
                for _ in range(8):
                    vs.append(plsc.load_gather(rows_v, [row_ids, hh]))
                    ts.append(plsc.load_gather(type_v, [tbase + hh]))
                    gs.append(plsc.load_gather(gamma_v, [hh]))
                    bs.append(plsc.load_gather(beta_v, [hh]))
                    hs.append(hh)
                    hh = (hh + 1) & (H - 1)
                for j in range(8):
                    o = (vs[j] + ts[j] - mean) * rstd * gs[j] + bs[j]
                    plsc.store_scatter(rows_v, [row_ids, hs[j]], o)
            return c2

        lax.fori_loop(0, CHUNK // 16, group_body, 0)
        pltpu.sync_copy(rows_v, out_hbm.at[pl.ds(start, CHUNK)])
        return carry

    lax.fori_loop(0, NCHUNK, chunk_body, 0)


@jax.jit
def _run(seq2d, tid_flat, seq_table, type_flat, gamma, beta):
    mesh = plsc.VectorSubcoreMesh(core_axis_name="c", subcore_axis_name="s")
    k = pl.kernel(
        _body,
        out_type=jax.ShapeDtypeStruct((BL, H), jnp.float32),
        mesh=mesh,
        scratch_types=[
            pltpu.VMEM((NSUB, 128), jnp.int32),
            pltpu.VMEM((CHUNK,), jnp.int32),
            pltpu.VMEM((CHUNK, H), jnp.float32),
            pltpu.VMEM((TYPE_VOCAB * H,), jnp.float32),
            pltpu.VMEM((H,), jnp.float32),
            pltpu.VMEM((H,), jnp.float32),
            pltpu.SemaphoreType.DMA,
        ],
        compiler_params=pltpu.CompilerParams(
            use_tc_tiling_on_sc=False,
            needs_layout_passes=False,
        ),
    )
    return k(seq2d, tid_flat, seq_table, type_flat, gamma, beta)


def kernel(seq, type_ids, seq_table, type_table, gamma, beta):
    seq2d = seq.astype(jnp.int32).reshape(BL // 128, 128)
    tid_flat = type_ids.astype(jnp.int32).reshape(BL)
    type_flat = type_table.reshape(TYPE_VOCAB * H)
    out = _run(seq2d, tid_flat, seq_table, type_flat, gamma, beta)
    return out.reshape(B, L, H)
